# Initial kernel scaffold; baseline (speedup 1.0000x reference)
import functools

import jax
import jax.numpy as jnp
import numpy as np
from jax import lax
from jax.experimental import pallas as pl
from jax.experimental.pallas import tpu as pltpu
from jax.experimental.pallas import tpu_sc as plsc

K = 64
ROWS = 128
N = 8192
NC = 2
NS = 16
NW = NC * NS
ROWS_PER_W = ROWS // NW
L = 16
NV = N // L
INT_MIN = jnp.int32(-(2 ** 31))


def _sc_body(x_hbm, mask_hbm, out_hbm, x_v, mask_v, key_v, cand_k, cand_i,
             sel_i, out_v):
    wid = lax.axis_index("s") * NC + lax.axis_index("c")

    def do_row(r, _):
        row = wid * ROWS_PER_W + r
        pltpu.sync_copy(x_hbm.at[row], x_v)
        pltpu.sync_copy(mask_hbm.at[row], mask_v)

        def p1(j, accs):
            accs = list(accs)
            for t in range(4):
                i = 4 * j + t
                xv = x_v[pl.ds(i * L, L)]
                mv = mask_v[pl.ds(i * L, L)]
                ik = plsc.bitcast(xv, jnp.int32)
                key = jnp.where(ik >= 0, ik, ik ^ jnp.int32(0x7FFFFFFF))
                key = jnp.where(mv == 0, INT_MIN, key)
                key_v[pl.ds(i * L, L)] = key
                accs[t] = jnp.maximum(accs[t], key)
            return tuple(accs)

        init = tuple(jnp.full((L,), INT_MIN, jnp.int32) for _ in range(4))
        a0, a1, a2, a3 = lax.fori_loop(0, NV // 4, p1, init)
        tlow = jnp.min(jnp.minimum(jnp.minimum(a0, a1), jnp.minimum(a2, a3)))

        def p2(i, off):
            kv = key_v[pl.ds(i * L, L)]
            selm = kv >= tlow
            iv = lax.iota(jnp.int32, L) + i * L
            plsc.store_compressed(cand_k.at[pl.ds(off, L)], kv, selm)
            plsc.store_compressed(cand_i.at[pl.ds(off, L)], iv, selm)
            return off + jnp.sum(selm.astype(jnp.int32))

        nc = lax.fori_loop(0, NV, p2, jnp.int32(0))
        cand_k[pl.ds(nc, L)] = jnp.full((L,), INT_MIN, jnp.int32)
        nvc = (nc + L - 1) // L

        def bit_step(b, m):
            t = m + lax.shift_left(jnp.int32(1), jnp.int32(31) - b)

            def cnt_body(i, acc):
                kv = cand_k[pl.ds(i * L, L)]
                return acc + jnp.sum((kv >= t).astype(jnp.int32))

            cnt = lax.fori_loop(0, nvc, cnt_body, jnp.int32(0))
            return jnp.where(cnt >= K, t, m)

        thr = lax.fori_loop(0, 32, bit_step, INT_MIN)

        def gt_body(i, acc):
            kv = cand_k[pl.ds(i * L, L)]
            return acc + jnp.sum((kv > thr).astype(jnp.int32))

        c_gt = lax.fori_loop(0, nvc, gt_body, jnp.int32(0))
        slots = K - c_gt

        def p4(i, carry):
            off, eqs = carry
            kv = cand_k[pl.ds(i * L, L)]
            iv = cand_i[pl.ds(i * L, L)]
            gt = kv > thr
            eq = kv == thr
            eqc = plsc.cumsum(eq.astype(jnp.int32))
            sel = gt | (eq & (eqc + eqs <= slots))
            plsc.store_compressed(sel_i.at[pl.ds(off, L)], iv, sel)
            return (off + jnp.sum(sel.astype(jnp.int32)),
                    eqs + jnp.sum(eq.astype(jnp.int32)))

        lax.fori_loop(0, nvc, p4, (jnp.int32(0), jnp.int32(0)))

        for j in range(K // L):
            idx = sel_i[pl.ds(j * L, L)]
            out_v[pl.ds(j * L, L)] = plsc.load_gather(x_v, [idx])
        pltpu.sync_copy(out_v, out_hbm.at[row])
        return 0

    lax.fori_loop(0, ROWS_PER_W, do_row, 0)


@jax.jit
def _kmax_sc(x, mask):
    mesh = plsc.VectorSubcoreMesh(core_axis_name="c", subcore_axis_name="s")
    return pl.kernel(
        _sc_body,
        out_type=jax.ShapeDtypeStruct((ROWS, K), jnp.float32),
        mesh=mesh,
        scratch_types=[
            pltpu.VMEM((N,), jnp.float32),
            pltpu.VMEM((N,), jnp.int32),
            pltpu.VMEM((N,), jnp.int32),
            pltpu.VMEM((N + L,), jnp.int32),
            pltpu.VMEM((N + L,), jnp.int32),
            pltpu.VMEM((K + L,), jnp.int32),
            pltpu.VMEM((K,), jnp.float32),
        ],
    )(x, mask)


def kernel(x, mask):
    return _kmax_sc(x, mask)


# speedup vs baseline: 2.7774x; 2.7774x over previous
import functools

import jax
import jax.numpy as jnp
import numpy as np
from jax import lax
from jax.experimental import pallas as pl
from jax.experimental.pallas import tpu as pltpu
from jax.experimental.pallas import tpu_sc as plsc

K = 64
ROWS = 128
N = 8192
NC = 2
NS = 16
NW = NC * NS
ROWS_PER_W = ROWS // NW
L = 16
NV = N // L
INT_MIN = np.int32(-(2 ** 31))
NEG_INF = np.float32(-np.inf)


def _sc_body(x_hbm, mask_hbm, out_hbm, x_v, mask_v, key_v, cand_k, cand_i,
             sel_i, out_v):
    wid = lax.axis_index("s") * NC + lax.axis_index("c")

    def do_row(r, _):
        row = wid * ROWS_PER_W + r
        pltpu.sync_copy(x_hbm.at[row], x_v)
        pltpu.sync_copy(mask_hbm.at[row], mask_v)

        def p1(j, accs):
            accs = list(accs)
            for t in range(4):
                i = 4 * j + t
                xv = x_v[pl.ds(i * L, L)]
                mv = mask_v[pl.ds(i * L, L)]
                ik = lax.bitcast_convert_type(xv, jnp.int32)
                key = jnp.where(ik >= 0, ik, ik ^ jnp.int32(0x7FFFFFFF))
                key = jnp.where(mv == 0, INT_MIN, key)
                key_v[pl.ds(i * L, L)] = key
                fv = jnp.where(mv == 0, NEG_INF, xv)
                accs[t] = jnp.maximum(accs[t], fv)
            return tuple(accs)

        init = tuple(jnp.full((L,), NEG_INF, jnp.float32) for _ in range(4))
        a0, a1, a2, a3 = lax.fori_loop(0, NV // 4, p1, init)
        vmin = jnp.minimum(jnp.minimum(a0, a1), jnp.minimum(a2, a3))
        tlow = -jnp.max(-vmin)

        def p2(i, off):
            kv = key_v[pl.ds(i * L, L)]
            xv = x_v[pl.ds(i * L, L)]
            mv = mask_v[pl.ds(i * L, L)]
            fv = jnp.where(mv == 0, NEG_INF, xv)
            selm = fv >= tlow
            iv = lax.iota(jnp.int32, L) + i * L
            plsc.store_compressed(cand_k.at[pl.ds(off, L)], kv, mask=selm)
            plsc.store_compressed(cand_i.at[pl.ds(off, L)], iv, mask=selm)
            return off + jnp.sum(selm.astype(jnp.int32))

        nc = lax.fori_loop(0, NV, p2, jnp.int32(0))
        cand_k[pl.ds(nc, L)] = jnp.full((L,), INT_MIN, jnp.int32)
        nvc = (nc + L - 1) // L

        def bit_step(b, m):
            t = m + lax.shift_left(jnp.int32(1), jnp.int32(31) - b)

            def cnt_body(i, acc):
                kv = cand_k[pl.ds(i * L, L)]
                return acc + jnp.sum((kv >= t).astype(jnp.int32))

            cnt = lax.fori_loop(0, nvc, cnt_body, jnp.int32(0))
            return jnp.where(cnt >= K, t, m)

        thr = lax.fori_loop(0, 32, bit_step, INT_MIN)

        def gt_body(i, acc):
            kv = cand_k[pl.ds(i * L, L)]
            return acc + jnp.sum((kv > thr).astype(jnp.int32))

        c_gt = lax.fori_loop(0, nvc, gt_body, jnp.int32(0))
        slots = K - c_gt

        def p4(i, carry):
            off, eqs = carry
            kv = cand_k[pl.ds(i * L, L)]
            iv = cand_i[pl.ds(i * L, L)]
            gt = kv > thr
            eq = kv == thr
            eqc = plsc.cumsum(eq.astype(jnp.int32))
            sel = gt | (eq & (eqc + eqs <= slots))
            plsc.store_compressed(sel_i.at[pl.ds(off, L)], iv, mask=sel)
            return (off + jnp.sum(sel.astype(jnp.int32)),
                    eqs + jnp.sum(eq.astype(jnp.int32)))

        lax.fori_loop(0, nvc, p4, (jnp.int32(0), jnp.int32(0)))

        for j in range(K // L):
            idx = sel_i[pl.ds(j * L, L)]
            out_v[pl.ds(j * L, L)] = plsc.load_gather(x_v, [idx])
        pltpu.sync_copy(out_v, out_hbm.at[row])
        return 0

    lax.fori_loop(0, ROWS_PER_W, do_row, 0)


@jax.jit
def _kmax_sc(x, mask):
    mesh = plsc.VectorSubcoreMesh(core_axis_name="c", subcore_axis_name="s")
    return pl.kernel(
        _sc_body,
        out_type=jax.ShapeDtypeStruct((ROWS, K), jnp.float32),
        mesh=mesh,
        compiler_params=pltpu.CompilerParams(needs_layout_passes=False),
        scratch_types=[
            pltpu.VMEM((N,), jnp.float32),
            pltpu.VMEM((N,), jnp.int32),
            pltpu.VMEM((N,), jnp.int32),
            pltpu.VMEM((N + L,), jnp.int32),
            pltpu.VMEM((N + L,), jnp.int32),
            pltpu.VMEM((K + L,), jnp.int32),
            pltpu.VMEM((K,), jnp.float32),
        ],
    )(x, mask)


def kernel(x, mask):
    return _kmax_sc(x, mask)


# speedup vs baseline: 2.9081x; 1.0471x over previous
import jax
import jax.numpy as jnp
import numpy as np
from jax import lax
from jax.experimental import pallas as pl
from jax.experimental.pallas import tpu as pltpu
from jax.experimental.pallas import tpu_sc as plsc

K = 64
ROWS = 128
N = 8192
NC = 2
NS = 16
NW = NC * NS
ROWS_PER_W = ROWS // NW
L = 16
NV = N // L
INT_MIN = np.int32(-(2 ** 31))
NEG_INF = np.float32(-np.inf)


def _lane0(v):
    return jnp.squeeze(lax.slice(v, (0,), (1,)), 0)


def _popcnt(m):
    return _lane0(plsc.all_reduce_population_count(m))


def _sc_body(x_hbm, mask_hbm, out_hbm, x_v, mask_v, cand_f, cand_k, cand_i,
             sel_i, out_v):
    wid = lax.axis_index("s") * NC + lax.axis_index("c")

    def do_row(r, _):
        row = wid * ROWS_PER_W + r
        pltpu.sync_copy(x_hbm.at[row], x_v)
        pltpu.sync_copy(mask_hbm.at[row], mask_v)

        def p1(j, accs):
            accs = list(accs)
            for t in range(4):
                i = 4 * j + t
                xv = x_v[pl.ds(i * L, L)]
                mv = mask_v[pl.ds(i * L, L)]
                fv = jnp.where(mv == 0, NEG_INF, xv)
                accs[t] = jnp.maximum(accs[t], fv)
            return tuple(accs)

        init = tuple(jnp.full((L,), NEG_INF, jnp.float32) for _ in range(4))
        a0, a1, a2, a3 = lax.fori_loop(0, NV // 4, p1, init)
        vmin = jnp.minimum(jnp.minimum(a0, a1), jnp.minimum(a2, a3))
        tlow = -jnp.max(-vmin)

        iota = lax.iota(jnp.int32, L)

        def p2(j, off):
            for t in range(4):
                i = 4 * j + t
                xv = x_v[pl.ds(i * L, L)]
                mv = mask_v[pl.ds(i * L, L)]
                fv = jnp.where(mv == 0, NEG_INF, xv)
                selm = fv >= tlow
                iv = iota + i * L
                plsc.store_compressed(cand_f.at[pl.ds(off, L)], fv, mask=selm)
                plsc.store_compressed(cand_i.at[pl.ds(off, L)], iv, mask=selm)
                off = off + _popcnt(selm)
            return off

        nc = lax.fori_loop(0, NV // 4, p2, jnp.int32(0))
        nvc = (nc + L - 1) // L

        def p2b(i, _unused):
            fv = cand_f[pl.ds(i * L, L)]
            ik = lax.bitcast_convert_type(fv, jnp.int32)
            cand_k[pl.ds(i * L, L)] = jnp.where(
                ik >= 0, ik, ik ^ jnp.int32(0x7FFFFFFF))
            return 0

        lax.fori_loop(0, nvc, p2b, 0)
        cand_k[pl.ds(nc, L)] = jnp.full((L,), INT_MIN, jnp.int32)

        def bit_step(b, m):
            t = m + lax.shift_left(jnp.int32(1), jnp.int32(31) - b)

            def cnt_body(i, acc):
                kv = cand_k[pl.ds(i * L, L)]
                return acc + (kv >= t).astype(jnp.int32)

            acc = lax.fori_loop(0, nvc, cnt_body, jnp.zeros((L,), jnp.int32))
            return jnp.where(jnp.sum(acc) >= K, t, m)

        thr = lax.fori_loop(0, 32, bit_step, INT_MIN)

        def gt_body(i, acc):
            kv = cand_k[pl.ds(i * L, L)]
            return acc + (kv > thr).astype(jnp.int32)

        gacc = lax.fori_loop(0, nvc, gt_body, jnp.zeros((L,), jnp.int32))
        slots = K - jnp.sum(gacc)

        def p4(i, carry):
            off, eqs = carry
            kv = cand_k[pl.ds(i * L, L)]
            iv = cand_i[pl.ds(i * L, L)]
            gt = kv > thr
            eq = kv == thr
            eqc = plsc.cumsum(eq.astype(jnp.int32))
            sel = gt | (eq & (eqc + eqs <= slots))
            plsc.store_compressed(sel_i.at[pl.ds(off, L)], iv, mask=sel)
            return (off + _popcnt(sel), eqs + _popcnt(eq))

        lax.fori_loop(0, nvc, p4, (jnp.int32(0), jnp.int32(0)))

        for j in range(K // L):
            idx = sel_i[pl.ds(j * L, L)]
            out_v[pl.ds(j * L, L)] = plsc.load_gather(x_v, [idx])
        pltpu.sync_copy(out_v, out_hbm.at[row])
        return 0

    lax.fori_loop(0, ROWS_PER_W, do_row, 0)


@jax.jit
def _kmax_sc(x, mask):
    mesh = plsc.VectorSubcoreMesh(core_axis_name="c", subcore_axis_name="s")
    return pl.kernel(
        _sc_body,
        out_type=jax.ShapeDtypeStruct((ROWS, K), jnp.float32),
        mesh=mesh,
        compiler_params=pltpu.CompilerParams(needs_layout_passes=False),
        scratch_types=[
            pltpu.VMEM((N,), jnp.float32),
            pltpu.VMEM((N,), jnp.int32),
            pltpu.VMEM((N + L,), jnp.float32),
            pltpu.VMEM((N + L,), jnp.int32),
            pltpu.VMEM((N + L,), jnp.int32),
            pltpu.VMEM((K + L,), jnp.int32),
            pltpu.VMEM((K,), jnp.float32),
        ],
    )(x, mask)


def kernel(x, mask):
    return _kmax_sc(x, mask)


# speedup vs baseline: 3.8821x; 1.3349x over previous
import jax
import jax.numpy as jnp
import numpy as np
from jax import lax
from jax.experimental import pallas as pl
from jax.experimental.pallas import tpu as pltpu
from jax.experimental.pallas import tpu_sc as plsc

K = 64
ROWS = 128
N = 8192
NC = 2
NS = 16
NW = NC * NS
ROWS_PER_W = ROWS // NW
L = 16
NV = N // L
INT_MIN = np.int32(-(2 ** 31))
NEG_INF = np.float32(-np.inf)


def _lane0(v):
    return jnp.squeeze(lax.slice(v, (0,), (1,)), 0)


def _popcnt(m):
    return _lane0(plsc.all_reduce_population_count(m))


def _keyvec(fv):
    ik = lax.bitcast_convert_type(fv, jnp.int32)
    return jnp.where(ik >= 0, ik, ik ^ jnp.int32(0x7FFFFFFF))


def _sc_body(x_hbm, mask_hbm, out_hbm, x_v0, x_v1, m_v0, m_v1, cand_f,
             cand_k, cand_i, sel_i, o_v0, o_v1, o_v2, o_v3, semx, semy):
    wid = lax.axis_index("s") * NC + lax.axis_index("c")
    row0 = wid * ROWS_PER_W
    xbufs = (x_v0, x_v1)
    mbufs = (m_v0, m_v1)
    obufs = (o_v0, o_v1, o_v2, o_v3)

    def issue(r, b):
        sem = semx if b == 0 else semy
        hx = pltpu.async_copy(x_hbm.at[row0 + r], xbufs[b], sem)
        hm = pltpu.async_copy(mask_hbm.at[row0 + r], mbufs[b], sem)
        return hx, hm

    pending = issue(0, 0)
    out_handles = []
    for r in range(ROWS_PER_W):
        b = r % 2
        pending[0].wait()
        pending[1].wait()
        if r + 1 < ROWS_PER_W:
            pending = issue(r + 1, 1 - b)
        xb = xbufs[b]
        mb = mbufs[b]

        def p1(j, accs):
            accs = list(accs)
            for t in range(4):
                i = 4 * j + t
                xv = xb[pl.ds(i * L, L)]
                mv = mb[pl.ds(i * L, L)]
                fv = jnp.where(mv == 0, NEG_INF, xv)
                accs[t] = jnp.maximum(accs[t], fv)
            return tuple(accs)

        init = tuple(jnp.full((L,), NEG_INF, jnp.float32) for _ in range(4))
        a0, a1, a2, a3 = lax.fori_loop(0, NV // 4, p1, init)
        vmin = jnp.minimum(jnp.minimum(a0, a1), jnp.minimum(a2, a3))
        vmax = jnp.maximum(jnp.maximum(a0, a1), jnp.maximum(a2, a3))
        tlow = -jnp.max(-vmin)
        tmax = jnp.max(vmax)

        iota = lax.iota(jnp.int32, L)

        def p2(j, off):
            fvs, selms, pcs = [], [], []
            for t in range(4):
                i = 4 * j + t
                xv = xb[pl.ds(i * L, L)]
                mv = mb[pl.ds(i * L, L)]
                fv = jnp.where(mv == 0, NEG_INF, xv)
                selm = fv >= tlow
                fvs.append(fv)
                selms.append(selm)
                pcs.append(_popcnt(selm))
            offs = [off]
            for t in range(3):
                offs.append(offs[-1] + pcs[t])
            for t in range(4):
                i = 4 * j + t
                iv = iota + i * L
                plsc.store_compressed(
                    cand_f.at[pl.ds(offs[t], L)], fvs[t], mask=selms[t])
                plsc.store_compressed(
                    cand_i.at[pl.ds(offs[t], L)], iv, mask=selms[t])
            return offs[3] + pcs[3]

        nc = lax.fori_loop(0, NV // 4, p2, jnp.int32(0))
        nvc = (nc + L - 1) // L

        def p2b(i, _unused):
            fv = cand_f[pl.ds(i * L, L)]
            cand_k[pl.ds(i * L, L)] = _keyvec(fv)
            return 0

        lax.fori_loop(0, nvc, p2b, 0)
        cand_k[pl.ds(nc, L)] = jnp.full((L,), INT_MIN, jnp.int32)

        lo0 = _lane0(_keyvec(jnp.full((L,), 0.0, jnp.float32) + tlow))
        hi0 = _lane0(_keyvec(jnp.full((L,), 0.0, jnp.float32) + tmax)) \
            + jnp.int32(1)

        def count_ge(t):
            def cnt_body(i, acc):
                kv = cand_k[pl.ds(i * L, L)]
                return acc + (kv >= t).astype(jnp.int32)

            acc = lax.fori_loop(0, nvc, cnt_body, jnp.zeros((L,), jnp.int32))
            return jnp.sum(acc)

        def bs_cond(c):
            lo, hi = c
            span = hi - lo
            return (span != 0) & (span != 1)

        def bs_body(c):
            lo, hi = c
            half = lax.shift_right_logical(hi - lo, 1)
            mid = lo + half
            ge = count_ge(mid) >= K
            return (jnp.where(ge, mid, lo), jnp.where(ge, hi, mid))

        thr, _ = lax.while_loop(bs_cond, bs_body, (lo0, hi0))

        def gt_body(i, acc):
            kv = cand_k[pl.ds(i * L, L)]
            return acc + (kv > thr).astype(jnp.int32)

        gacc = lax.fori_loop(0, nvc, gt_body, jnp.zeros((L,), jnp.int32))
        slots = K - jnp.sum(gacc)

        def p4(i, carry):
            off, eqs = carry
            kv = cand_k[pl.ds(i * L, L)]
            iv = cand_i[pl.ds(i * L, L)]
            gt = kv > thr
            eq = kv == thr
            eqc = plsc.cumsum(eq.astype(jnp.int32))
            sel = gt | (eq & (eqc + eqs <= slots))
            plsc.store_compressed(sel_i.at[pl.ds(off, L)], iv, mask=sel)
            return (off + _popcnt(sel), eqs + _popcnt(eq))

        lax.fori_loop(0, nvc, p4, (jnp.int32(0), jnp.int32(0)))

        ob = obufs[r]
        for j in range(K // L):
            idx = sel_i[pl.ds(j * L, L)]
            ob[pl.ds(j * L, L)] = plsc.load_gather(xb, [idx])
        out_handles.append(
            pltpu.async_copy(ob, out_hbm.at[row0 + r], semx if b else semy))

    for h in out_handles:
        h.wait()


@jax.jit
def _kmax_sc(x, mask):
    mesh = plsc.VectorSubcoreMesh(core_axis_name="c", subcore_axis_name="s")
    return pl.kernel(
        _sc_body,
        out_type=jax.ShapeDtypeStruct((ROWS, K), jnp.float32),
        mesh=mesh,
        compiler_params=pltpu.CompilerParams(needs_layout_passes=False),
        scratch_types=[
            pltpu.VMEM((N,), jnp.float32),
            pltpu.VMEM((N,), jnp.float32),
            pltpu.VMEM((N,), jnp.int32),
            pltpu.VMEM((N,), jnp.int32),
            pltpu.VMEM((N + L,), jnp.float32),
            pltpu.VMEM((N + L,), jnp.int32),
            pltpu.VMEM((N + L,), jnp.int32),
            pltpu.VMEM((K + L,), jnp.int32),
            pltpu.VMEM((K,), jnp.float32),
            pltpu.VMEM((K,), jnp.float32),
            pltpu.VMEM((K,), jnp.float32),
            pltpu.VMEM((K,), jnp.float32),
            pltpu.SemaphoreType.DMA,
            pltpu.SemaphoreType.DMA,
        ],
    )(x, mask)


def kernel(x, mask):
    return _kmax_sc(x, mask)


# speedup vs baseline: 4.9229x; 1.2681x over previous
import jax
import jax.numpy as jnp
import numpy as np
from jax import lax
from jax.experimental import pallas as pl
from jax.experimental.pallas import tpu as pltpu
from jax.experimental.pallas import tpu_sc as plsc

K = 64
ROWS = 128
N = 8192
NC = 2
NS = 16
NW = NC * NS
ROWS_PER_W = ROWS // NW
L = 16
NV = N // L
INT_MIN = np.int32(-(2 ** 31))
NEG_INF = np.float32(-np.inf)


def _lane0(v):
    return jnp.squeeze(lax.slice(v, (0,), (1,)), 0)


def _popcnt(m):
    return _lane0(plsc.all_reduce_population_count(m))


def _keyvec(fv):
    ik = lax.bitcast_convert_type(fv, jnp.int32)
    return jnp.where(ik >= 0, ik, ik ^ jnp.int32(0x7FFFFFFF))


def _sc_body(x_hbm, mask_hbm, out_hbm, x_v0, x_v1, m_v0, m_v1, cand_f,
             cand_k, cand_i, sel_i, o_v0, o_v1, o_v2, o_v3, semx, semy):
    wid = lax.axis_index("s") * NC + lax.axis_index("c")
    row0 = wid * ROWS_PER_W
    xbufs = (x_v0, x_v1)
    mbufs = (m_v0, m_v1)
    obufs = (o_v0, o_v1, o_v2, o_v3)

    def issue(r, b):
        sem = semx if b == 0 else semy
        hx = pltpu.async_copy(x_hbm.at[row0 + r], xbufs[b], sem)
        hm = pltpu.async_copy(mask_hbm.at[row0 + r], mbufs[b], sem)
        return hx, hm

    pending = issue(0, 0)
    out_handles = []
    for r in range(ROWS_PER_W):
        b = r % 2
        pending[0].wait()
        pending[1].wait()
        if r + 1 < ROWS_PER_W:
            pending = issue(r + 1, 1 - b)
        xb = xbufs[b]
        mb = mbufs[b]

        def p1(j, accs):
            accs = list(accs)
            for t in range(4):
                i = 4 * j + t
                xv = xb[pl.ds(i * L, L)]
                mv = mb[pl.ds(i * L, L)]
                fv = jnp.where(mv == 0, NEG_INF, xv)
                accs[t] = jnp.maximum(accs[t], fv)
            return tuple(accs)

        init = tuple(jnp.full((L,), NEG_INF, jnp.float32) for _ in range(4))
        a0, a1, a2, a3 = lax.fori_loop(0, NV // 4, p1, init)
        vmin = jnp.minimum(jnp.minimum(a0, a1), jnp.minimum(a2, a3))
        vmax = jnp.maximum(jnp.maximum(a0, a1), jnp.maximum(a2, a3))
        tlow = -jnp.max(-vmin)
        tmax = jnp.max(vmax)

        iota = lax.iota(jnp.int32, L)

        def p2(j, off):
            fvs, selms, pcs = [], [], []
            for t in range(8):
                i = 8 * j + t
                xv = xb[pl.ds(i * L, L)]
                mv = mb[pl.ds(i * L, L)]
                fv = jnp.where(mv == 0, NEG_INF, xv)
                selm = fv >= tlow
                fvs.append(fv)
                selms.append(selm)
                pcs.append(_popcnt(selm))
            offs = [off]
            for t in range(7):
                offs.append(offs[-1] + pcs[t])
            for t in range(8):
                i = 8 * j + t
                iv = iota + i * L
                plsc.store_compressed(
                    cand_f.at[pl.ds(offs[t], L)], fvs[t], mask=selms[t])
                plsc.store_compressed(
                    cand_i.at[pl.ds(offs[t], L)], iv, mask=selms[t])
            return offs[7] + pcs[7]

        nc = lax.fori_loop(0, NV // 8, p2, jnp.int32(0))
        nvc = (nc + L - 1) // L

        def p2b(i, _unused):
            fv = cand_f[pl.ds(i * L, L)]
            cand_k[pl.ds(i * L, L)] = _keyvec(fv)
            return 0

        lax.fori_loop(0, nvc, p2b, 0)
        cand_k[pl.ds(nc, L)] = jnp.full((L,), INT_MIN, jnp.int32)

        lo0 = _lane0(_keyvec(jnp.full((L,), 0.0, jnp.float32) + tlow))
        hi0 = _lane0(_keyvec(jnp.full((L,), 0.0, jnp.float32) + tmax)) \
            + jnp.int32(1)

        def bs_cond(c):
            lo, hi, _ = c
            span = hi - lo
            return (span != 0) & (span != 1)

        def bs_body(c):
            lo, hi, chi = c
            span = hi - lo
            h = lax.shift_right_logical(span, 1)
            q = lax.shift_right_logical(span, 2)
            p1 = lo + q
            p2_ = lo + h
            p3 = lo + h + q

            def cnt_body(i, accs):
                a1, a2, a3 = accs
                kv = cand_k[pl.ds(i * L, L)]
                return (a1 + (kv >= p1).astype(jnp.int32),
                        a2 + (kv >= p2_).astype(jnp.int32),
                        a3 + (kv >= p3).astype(jnp.int32))

            z = jnp.zeros((L,), jnp.int32)
            a1, a2, a3 = lax.fori_loop(0, nvc, cnt_body, (z, z, z))
            c1 = jnp.sum(a1)
            c2 = jnp.sum(a2)
            c3 = jnp.sum(a3)
            g1 = c1 >= K
            g2 = c2 >= K
            g3 = c3 >= K
            nlo = jnp.where(g3, p3, jnp.where(g2, p2_, jnp.where(g1, p1, lo)))
            nhi = jnp.where(g3, hi, jnp.where(g2, p3, jnp.where(g1, p2_, p1)))
            nchi = jnp.where(g3, chi, jnp.where(g2, c3, jnp.where(g1, c2, c1)))
            return (nlo, nhi, nchi)

        thr, _, c_gt = lax.while_loop(bs_cond, bs_body,
                                      (lo0, hi0, jnp.int32(0)))
        slots = K - c_gt

        def p4(i, carry):
            off, eqs = carry
            kv = cand_k[pl.ds(i * L, L)]
            iv = cand_i[pl.ds(i * L, L)]
            gt = kv > thr
            eq = kv == thr
            eqc = plsc.cumsum(eq.astype(jnp.int32))
            sel = gt | (eq & (eqc + eqs <= slots))
            plsc.store_compressed(sel_i.at[pl.ds(off, L)], iv, mask=sel)
            return (off + _popcnt(sel), eqs + _popcnt(eq))

        lax.fori_loop(0, nvc, p4, (jnp.int32(0), jnp.int32(0)))

        ob = obufs[r]
        for j in range(K // L):
            idx = sel_i[pl.ds(j * L, L)]
            ob[pl.ds(j * L, L)] = plsc.load_gather(xb, [idx])
        out_handles.append(
            pltpu.async_copy(ob, out_hbm.at[row0 + r], semx if b else semy))

    for h in out_handles:
        h.wait()


@jax.jit
def _kmax_sc(x, mask):
    mesh = plsc.VectorSubcoreMesh(core_axis_name="c", subcore_axis_name="s")
    return pl.kernel(
        _sc_body,
        out_type=jax.ShapeDtypeStruct((ROWS, K), jnp.float32),
        mesh=mesh,
        compiler_params=pltpu.CompilerParams(needs_layout_passes=False),
        scratch_types=[
            pltpu.VMEM((N,), jnp.float32),
            pltpu.VMEM((N,), jnp.float32),
            pltpu.VMEM((N,), jnp.int32),
            pltpu.VMEM((N,), jnp.int32),
            pltpu.VMEM((N + L,), jnp.float32),
            pltpu.VMEM((N + L,), jnp.int32),
            pltpu.VMEM((N + L,), jnp.int32),
            pltpu.VMEM((K + L,), jnp.int32),
            pltpu.VMEM((K,), jnp.float32),
            pltpu.VMEM((K,), jnp.float32),
            pltpu.VMEM((K,), jnp.float32),
            pltpu.VMEM((K,), jnp.float32),
            pltpu.SemaphoreType.DMA,
            pltpu.SemaphoreType.DMA,
        ],
    )(x, mask)


def kernel(x, mask):
    return _kmax_sc(x, mask)
